# baseline (speedup 1.0000x reference)
import functools

import jax
import jax.numpy as jnp
from jax import lax
from jax.experimental import pallas as pl
from jax.experimental.pallas import tpu as pltpu
from jax.experimental.pallas import tpu_sc as plsc

_N = 10000
_E = 320000
_D = 128
_B = 64
_H = 128
_OUT = 36

_NC = 2
_NS = 16
_HF = _H // 2
_CH = 128
_KW = 159
_ET = _KW * _CH
_EPAD = _NS * _ET
_NP = 10240
_RPS = _NP // _NS
_RB = 1024
_NBLK = _NP // _RB

_MESH = plsc.VectorSubcoreMesh(core_axis_name="c", subcore_axis_name="s")


@functools.partial(
    pl.kernel,
    out_type=jax.ShapeDtypeStruct((_NC, _NP), jnp.float32),
    mesh=_MESH,
    scratch_types=[
        pltpu.VMEM((_KW, _CH), jnp.int32),
        pltpu.VMEM((_ET,), jnp.float32),
        pltpu.VMEM_SHARED((_NP,), jnp.float32),
        pltpu.SemaphoreType.DMA,
    ],
)
def _deg_kernel(dstr, ewr, zn, out, dst_v, ew_v, acc, sem):
    c = lax.axis_index("c")
    s = lax.axis_index("s")
    pltpu.sync_copy(zn.at[pl.ds(s * _RPS, _RPS)], acc.at[pl.ds(s * _RPS, _RPS)])
    pltpu.sync_copy(dstr.at[s], dst_v)
    pltpu.sync_copy(ewr.at[s], ew_v)
    plsc.subcore_barrier()

    def body(j, carry):
        pltpu.async_copy(ew_v.at[pl.ds(j * _CH, _CH)], acc.at[dst_v.at[j]], sem,
                         add=True)
        return carry

    def drain(j, carry):
        pltpu.make_async_copy(
            ew_v.at[pl.ds(j * _CH, _CH)], acc.at[dst_v.at[j]], sem).wait()
        return carry

    lax.fori_loop(80 * c, 80 + 79 * c, body, 0)
    lax.fori_loop(80 * c, 80 + 79 * c, drain, 0)
    plsc.subcore_barrier()
    pltpu.sync_copy(acc.at[pl.ds(s * _RPS, _RPS)], out.at[c, pl.ds(s * _RPS, _RPS)])


@functools.partial(
    pl.kernel,
    out_type=jax.ShapeDtypeStruct((_NP, _H), jnp.float32),
    mesh=_MESH,
    scratch_types=[
        pltpu.VMEM((_ET,), jnp.int32),
        pltpu.VMEM((_KW, _CH), jnp.int32),
        pltpu.VMEM((_ET,), jnp.float32),
        pltpu.VMEM((3, _CH, _HF), jnp.float32),
        pltpu.VMEM_SHARED((_NP, _HF), jnp.float32),
        pltpu.SemaphoreType.DMA,
        pltpu.SemaphoreType.DMA,
        pltpu.SemaphoreType.DMA,
        pltpu.SemaphoreType.DMA,
        pltpu.SemaphoreType.DMA,
        pltpu.SemaphoreType.DMA,
    ],
    compiler_params=pltpu.CompilerParams(use_tc_tiling_on_sc=False),
)
def _agg_kernel(g2, srcr, dstr, ewr, znd, out, src_v, dst_v, ew_v, rows_v, acc,
                gs0, gs1, gs2, ss0, ss1, ss2):
    gsems = (gs0, gs1, gs2)
    ssems = (ss0, ss1, ss2)
    c = lax.axis_index("c")
    s = lax.axis_index("s")
    pltpu.sync_copy(znd.at[pl.ds(s * _RPS, _RPS)], acc.at[pl.ds(s * _RPS, _RPS)])
    pltpu.sync_copy(srcr.at[s], src_v)
    pltpu.sync_copy(dstr.at[s], dst_v)
    pltpu.sync_copy(ewr.at[s], ew_v)
    @plsc.parallel_loop(0, _ET, step=16, unroll=4)
    def fold(i):
        src_v[pl.ds(i, 16)] = src_v[pl.ds(i, 16)] * 2 + c

    plsc.subcore_barrier()

    dn = lax.GatherDimensionNumbers(
        offset_dims=(), collapsed_slice_dims=(0,), start_index_map=(0,))

    def scale(k, j):
        @plsc.parallel_loop(0, _CH, step=16, unroll=4)
        def group(gbase):
            ewg = ew_v[pl.ds(j * _CH + gbase, 16)]
            sps = [
                lax.gather(ewg, jnp.full((16, 1), l, jnp.int32), dn, (1,),
                           mode=lax.GatherScatterMode.PROMISE_IN_BOUNDS)
                for l in range(16)
            ]
            for l in range(16):
                r = gbase + l
                for f in range(_HF // 16):
                    rows_v[k, r, pl.ds(f * 16, 16)] = (
                        rows_v[k, r, pl.ds(f * 16, 16)] * sps[l])

    def gidx_ref(j):
        return src_v.at[pl.ds(j * _CH, _CH)]

    for k in range(2):
        pltpu.async_copy(g2.at[gidx_ref(k)], rows_v.at[k], gsems[k], priority=1)

    def sup(gi, carry):
        for k in range(3):
            j = gi * 3 + k
            kn = (k + 2) % 3
            pltpu.make_async_copy(g2.at[gidx_ref(j)], rows_v.at[k], gsems[k]).wait()
            scale(k, j)
            pltpu.async_copy(rows_v.at[k], acc.at[dst_v.at[j]], ssems[k], add=True)

            @pl.when(j >= 1)
            def _w(j=j, kn=kn):
                pltpu.make_async_copy(
                    rows_v.at[kn], acc.at[dst_v.at[j - 1]], ssems[kn]).wait()

            @pl.when(j + 2 < _KW)
            def _g(j=j, kn=kn):
                pltpu.async_copy(g2.at[gidx_ref(j + 2)], rows_v.at[kn], gsems[kn],
                                 priority=1)
        return carry

    lax.fori_loop(0, _KW // 3, sup, 0)
    k = (_KW - 1) % 3
    pltpu.make_async_copy(rows_v.at[k], acc.at[dst_v.at[_KW - 1]], ssems[k]).wait()
    plsc.subcore_barrier()
    pltpu.sync_copy(acc.at[pl.ds(s * _RPS, _RPS)],
                    out.at[pl.ds(s * _RPS, _RPS), pl.ds(c * _HF, _HF)])


def _tc1_body(x_ref, degp_ref, w1_ref, g_ref):
    deg = degp_ref[0] + degp_ref[1] + 1.0
    dinv = lax.rsqrt(deg)
    hw = jnp.dot(x_ref[...], w1_ref[...], preferred_element_type=jnp.float32)
    g_ref[...] = hw * dinv


_tc1 = pl.pallas_call(
    _tc1_body,
    grid=(_NBLK,),
    in_specs=[
        pl.BlockSpec((_RB, _D), lambda i: (i, 0)),
        pl.BlockSpec((_NC, _RB, 1), lambda i: (0, i, 0)),
        pl.BlockSpec((_D, _H), lambda i: (0, 0)),
    ],
    out_specs=pl.BlockSpec((_RB, _H), lambda i: (i, 0)),
    out_shape=jax.ShapeDtypeStruct((_NP, _H), jnp.float32),
)


def _tc2_body(aggp_ref, g1_ref, degp_ref, w2_ref, b1_ref, g2_ref):
    deg = degp_ref[0] + degp_ref[1] + 1.0
    dinv = lax.rsqrt(deg)
    h1 = jnp.maximum(dinv * (aggp_ref[...] + g1_ref[...]) + b1_ref[...], 0.0)
    hw2 = jnp.dot(h1, w2_ref[...], preferred_element_type=jnp.float32)
    g2_ref[...] = hw2 * dinv


_tc2 = pl.pallas_call(
    _tc2_body,
    grid=(_NBLK,),
    in_specs=[
        pl.BlockSpec((_RB, _H), lambda i: (i, 0)),
        pl.BlockSpec((_RB, _H), lambda i: (i, 0)),
        pl.BlockSpec((_NC, _RB, 1), lambda i: (0, i, 0)),
        pl.BlockSpec((_H, _H), lambda i: (0, 0)),
        pl.BlockSpec((1, _H), lambda i: (0, 0)),
    ],
    out_specs=pl.BlockSpec((_RB, _H), lambda i: (i, 0)),
    out_shape=jax.ShapeDtypeStruct((_NP, _H), jnp.float32),
)


def _tc3_body(aggp_ref, g2_ref, degp_ref, batch_ref, b2_ref, rho_ref,
              m1a_ref, m1b_ref, m1c_ref, mb1_ref, m2_ref, mb2_ref, m3_ref, mb3_ref,
              out_ref, ssum, smax, scnt):
    i = pl.program_id(0)

    @pl.when(i == 0)
    def _init():
        ssum[...] = jnp.zeros_like(ssum)
        smax[...] = jnp.full_like(smax, -jnp.inf)
        scnt[...] = jnp.zeros_like(scnt)

    deg = degp_ref[0] + degp_ref[1] + 1.0
    dinv = lax.rsqrt(deg)
    h2 = jnp.maximum(dinv * (aggp_ref[...] + g2_ref[...]) + b2_ref[...], 0.0)

    bt = batch_ref[...]
    oh = (bt == lax.broadcasted_iota(jnp.int32, (1, _B), 1)).astype(jnp.float32)
    dn = (((0,), (0,)), ((), ()))
    ssum[...] += lax.dot_general(oh, h2, dn, preferred_element_type=jnp.float32)
    scnt[...] += lax.dot_general(oh, jnp.ones((_RB, 1), jnp.float32), dn,
                                 preferred_element_type=jnp.float32)

    bmin = jnp.min(bt)
    bmax = jnp.max(bt)
    for b in range(_B):
        @pl.when((bmin <= b) & (b <= bmax))
        def _seg(b=b):
            m = bt == b
            contrib = jnp.max(jnp.where(m, h2, -jnp.inf), axis=0, keepdims=True)
            smax[b:b + 1, :] = jnp.maximum(smax[b:b + 1, :], contrib)

    @pl.when(i == pl.num_programs(0) - 1)
    def _fin():
        gmp = smax[...]
        gmp = jnp.where(gmp == -jnp.inf, 0.0, gmp)
        gap = ssum[...] / jnp.maximum(scnt[...], 1.0)
        z = (jnp.dot(gmp, m1a_ref[...], preferred_element_type=jnp.float32)
             + jnp.dot(gap, m1b_ref[...], preferred_element_type=jnp.float32)
             + jnp.dot(rho_ref[...], m1c_ref[...], preferred_element_type=jnp.float32)
             + mb1_ref[...])
        z = jnp.maximum(z, 0.0)
        z = jnp.maximum(jnp.dot(z, m2_ref[...], preferred_element_type=jnp.float32)
                        + mb2_ref[...], 0.0)
        out_ref[...] = (jnp.dot(z, m3_ref[...], preferred_element_type=jnp.float32)
                        + mb3_ref[...])


_tc3 = pl.pallas_call(
    _tc3_body,
    grid=(_NBLK,),
    in_specs=[
        pl.BlockSpec((_RB, _H), lambda i: (i, 0)),
        pl.BlockSpec((_RB, _H), lambda i: (i, 0)),
        pl.BlockSpec((_NC, _RB, 1), lambda i: (0, i, 0)),
        pl.BlockSpec((_RB, 1), lambda i: (i, 0)),
        pl.BlockSpec((1, _H), lambda i: (0, 0)),
        pl.BlockSpec((_B, 1), lambda i: (0, 0)),
        pl.BlockSpec((_H, _H), lambda i: (0, 0)),
        pl.BlockSpec((_H, _H), lambda i: (0, 0)),
        pl.BlockSpec((1, _H), lambda i: (0, 0)),
        pl.BlockSpec((1, _H), lambda i: (0, 0)),
        pl.BlockSpec((_H, _H), lambda i: (0, 0)),
        pl.BlockSpec((1, _H), lambda i: (0, 0)),
        pl.BlockSpec((_H, _OUT), lambda i: (0, 0)),
        pl.BlockSpec((1, _OUT), lambda i: (0, 0)),
    ],
    out_specs=pl.BlockSpec((_B, _OUT), lambda i: (0, 0)),
    out_shape=jax.ShapeDtypeStruct((_B, _OUT), jnp.float32),
    scratch_shapes=[
        pltpu.VMEM((_B, _H), jnp.float32),
        pltpu.VMEM((_B, _H), jnp.float32),
        pltpu.VMEM((_B, 1), jnp.float32),
    ],
)


def kernel(x, edge_index, edge_attr, batch, rho, W1, b1, W2, b2,
           M1, mb1, M2, mb2, M3, mb3):
    xp = jnp.pad(x, ((0, _NP - _N), (0, 0)))
    batch_p = jnp.pad(batch, (0, _NP - _N), constant_values=_B).reshape(_NP, 1)
    pad = _EPAD - _E
    apad = jnp.arange(pad, dtype=jnp.int32)
    src = jnp.concatenate([edge_index[0], apad % _N])
    dst = jnp.concatenate([edge_index[1], _N + apad % (_NP - _N)])
    ew = jnp.concatenate([edge_attr, jnp.zeros((pad,), jnp.float32)])
    srcr = src.reshape(_NS, _ET)
    dstr = dst.reshape(_NS, _KW, _CH)
    ewr = ew.reshape(_NS, _ET)
    zn = jnp.zeros((_NP,), jnp.float32)
    znd = jnp.zeros((_NP, _HF), jnp.float32)
    rho_c = rho.reshape(_B, 1)
    m1a = M1[:_H]
    m1b = M1[_H:2 * _H]
    m1c = M1[2 * _H:]

    degp = _deg_kernel(dstr, ewr, zn).reshape(_NC, _NP, 1)
    g1 = _tc1(xp, degp, W1)
    aggp1 = _agg_kernel(g1.reshape(_NC * _NP, _HF), srcr, dstr, ewr, znd)
    g2 = _tc2(aggp1, g1, degp, W2, b1.reshape(1, _H))
    aggp2 = _agg_kernel(g2.reshape(_NC * _NP, _HF), srcr, dstr, ewr, znd)
    return _tc3(aggp2, g2, degp, batch_p, b2.reshape(1, _H), rho_c,
                m1a, m1b, m1c, mb1.reshape(1, _H), M2, mb2.reshape(1, _H),
                M3, mb3.reshape(1, _OUT))


# speedup vs baseline: 1.0023x; 1.0023x over previous
import functools

import jax
import jax.numpy as jnp
from jax import lax
from jax.experimental import pallas as pl
from jax.experimental.pallas import tpu as pltpu
from jax.experimental.pallas import tpu_sc as plsc

_N = 10000
_E = 320000
_D = 128
_B = 64
_H = 128
_OUT = 36

_NC = 2
_NS = 16
_HF = _H // 2
_CH = 128
_KW = 159
_ET = _KW * _CH
_EPAD = _NS * _ET
_NP = 10240
_RPS = _NP // _NS
_RB = 1024
_NBLK = _NP // _RB

_MESH = plsc.VectorSubcoreMesh(core_axis_name="c", subcore_axis_name="s")


@functools.partial(
    pl.kernel,
    out_type=jax.ShapeDtypeStruct((_NC, _NP), jnp.float32),
    mesh=_MESH,
    scratch_types=[
        pltpu.VMEM((_KW, _CH), jnp.int32),
        pltpu.VMEM((_ET,), jnp.float32),
        pltpu.VMEM_SHARED((_NP,), jnp.float32),
        pltpu.SemaphoreType.DMA,
    ],
)
def _deg_kernel(dstr, ewr, zn, out, dst_v, ew_v, acc, sem):
    c = lax.axis_index("c")
    s = lax.axis_index("s")
    pltpu.sync_copy(zn.at[pl.ds(s * _RPS, _RPS)], acc.at[pl.ds(s * _RPS, _RPS)])
    pltpu.sync_copy(dstr.at[s], dst_v)
    pltpu.sync_copy(ewr.at[s], ew_v)
    plsc.subcore_barrier()

    def body(j, carry):
        pltpu.async_copy(ew_v.at[pl.ds(j * _CH, _CH)], acc.at[dst_v.at[j]], sem,
                         add=True)
        return carry

    def drain(j, carry):
        pltpu.make_async_copy(
            ew_v.at[pl.ds(j * _CH, _CH)], acc.at[dst_v.at[j]], sem).wait()
        return carry

    lax.fori_loop(80 * c, 80 + 79 * c, body, 0)
    lax.fori_loop(80 * c, 80 + 79 * c, drain, 0)
    plsc.subcore_barrier()
    pltpu.sync_copy(acc.at[pl.ds(s * _RPS, _RPS)], out.at[c, pl.ds(s * _RPS, _RPS)])


@functools.partial(
    pl.kernel,
    out_type=jax.ShapeDtypeStruct((_NP, _H), jnp.float32),
    mesh=_MESH,
    scratch_types=[
        pltpu.VMEM((_ET,), jnp.int32),
        pltpu.VMEM((_KW, _CH), jnp.int32),
        pltpu.VMEM((_ET,), jnp.float32),
        pltpu.VMEM((3, _CH, _HF), jnp.float32),
        pltpu.VMEM_SHARED((_NP, _HF), jnp.float32),
        pltpu.SemaphoreType.DMA,
        pltpu.SemaphoreType.DMA,
        pltpu.SemaphoreType.DMA,
        pltpu.SemaphoreType.DMA,
        pltpu.SemaphoreType.DMA,
        pltpu.SemaphoreType.DMA,
    ],
    compiler_params=pltpu.CompilerParams(use_tc_tiling_on_sc=False),
)
def _agg_kernel(g2, srcr, dstr, ewr, znd, out, src_v, dst_v, ew_v, rows_v, acc,
                gs0, gs1, gs2, ss0, ss1, ss2):
    gsems = (gs0, gs1, gs2)
    ssems = (ss0, ss1, ss2)
    c = lax.axis_index("c")
    s = lax.axis_index("s")
    pltpu.sync_copy(znd.at[pl.ds(s * _RPS, _RPS)], acc.at[pl.ds(s * _RPS, _RPS)])
    pltpu.sync_copy(srcr.at[s], src_v)
    pltpu.sync_copy(dstr.at[s], dst_v)
    pltpu.sync_copy(ewr.at[s], ew_v)
    @plsc.parallel_loop(0, _ET, step=16, unroll=4)
    def fold(i):
        src_v[pl.ds(i, 16)] = src_v[pl.ds(i, 16)] * 2 + c

    plsc.subcore_barrier()

    dn = lax.GatherDimensionNumbers(
        offset_dims=(), collapsed_slice_dims=(0,), start_index_map=(0,))

    def scale(k, j):
        @plsc.parallel_loop(0, _CH, step=16, unroll=2)
        def group(gbase):
            ewg = ew_v[pl.ds(j * _CH + gbase, 16)]
            sps = [
                lax.gather(ewg, jnp.full((16, 1), l, jnp.int32), dn, (1,),
                           mode=lax.GatherScatterMode.PROMISE_IN_BOUNDS)
                for l in range(16)
            ]
            for l in range(16):
                r = gbase + l
                for f in range(_HF // 16):
                    rows_v[k, r, pl.ds(f * 16, 16)] = (
                        rows_v[k, r, pl.ds(f * 16, 16)] * sps[l])

    def gidx_ref(j):
        return src_v.at[pl.ds(j * _CH, _CH)]

    for k in range(2):
        pltpu.async_copy(g2.at[gidx_ref(k)], rows_v.at[k], gsems[k], priority=1)

    def sup(gi, carry):
        for k in range(3):
            j = gi * 3 + k
            kn = (k + 2) % 3
            pltpu.make_async_copy(g2.at[gidx_ref(j)], rows_v.at[k], gsems[k]).wait()
            scale(k, j)
            pltpu.async_copy(rows_v.at[k], acc.at[dst_v.at[j]], ssems[k], add=True)

            @pl.when(j >= 1)
            def _w(j=j, kn=kn):
                pltpu.make_async_copy(
                    rows_v.at[kn], acc.at[dst_v.at[j - 1]], ssems[kn]).wait()

            @pl.when(j + 2 < _KW)
            def _g(j=j, kn=kn):
                pltpu.async_copy(g2.at[gidx_ref(j + 2)], rows_v.at[kn], gsems[kn],
                                 priority=1)
        return carry

    lax.fori_loop(0, _KW // 3, sup, 0)
    k = (_KW - 1) % 3
    pltpu.make_async_copy(rows_v.at[k], acc.at[dst_v.at[_KW - 1]], ssems[k]).wait()
    plsc.subcore_barrier()
    pltpu.sync_copy(acc.at[pl.ds(s * _RPS, _RPS)],
                    out.at[pl.ds(s * _RPS, _RPS), pl.ds(c * _HF, _HF)])


def _tc1_body(x_ref, degp_ref, w1_ref, g_ref):
    deg = degp_ref[0] + degp_ref[1] + 1.0
    dinv = lax.rsqrt(deg)
    hw = jnp.dot(x_ref[...], w1_ref[...], preferred_element_type=jnp.float32)
    g_ref[...] = hw * dinv


_tc1 = pl.pallas_call(
    _tc1_body,
    grid=(_NBLK,),
    in_specs=[
        pl.BlockSpec((_RB, _D), lambda i: (i, 0)),
        pl.BlockSpec((_NC, _RB, 1), lambda i: (0, i, 0)),
        pl.BlockSpec((_D, _H), lambda i: (0, 0)),
    ],
    out_specs=pl.BlockSpec((_RB, _H), lambda i: (i, 0)),
    out_shape=jax.ShapeDtypeStruct((_NP, _H), jnp.float32),
)


def _tc2_body(aggp_ref, g1_ref, degp_ref, w2_ref, b1_ref, g2_ref):
    deg = degp_ref[0] + degp_ref[1] + 1.0
    dinv = lax.rsqrt(deg)
    h1 = jnp.maximum(dinv * (aggp_ref[...] + g1_ref[...]) + b1_ref[...], 0.0)
    hw2 = jnp.dot(h1, w2_ref[...], preferred_element_type=jnp.float32)
    g2_ref[...] = hw2 * dinv


_tc2 = pl.pallas_call(
    _tc2_body,
    grid=(_NBLK,),
    in_specs=[
        pl.BlockSpec((_RB, _H), lambda i: (i, 0)),
        pl.BlockSpec((_RB, _H), lambda i: (i, 0)),
        pl.BlockSpec((_NC, _RB, 1), lambda i: (0, i, 0)),
        pl.BlockSpec((_H, _H), lambda i: (0, 0)),
        pl.BlockSpec((1, _H), lambda i: (0, 0)),
    ],
    out_specs=pl.BlockSpec((_RB, _H), lambda i: (i, 0)),
    out_shape=jax.ShapeDtypeStruct((_NP, _H), jnp.float32),
)


def _tc3_body(aggp_ref, g2_ref, degp_ref, batch_ref, b2_ref, rho_ref,
              m1a_ref, m1b_ref, m1c_ref, mb1_ref, m2_ref, mb2_ref, m3_ref, mb3_ref,
              out_ref, ssum, smax, scnt):
    i = pl.program_id(0)

    @pl.when(i == 0)
    def _init():
        ssum[...] = jnp.zeros_like(ssum)
        smax[...] = jnp.full_like(smax, -jnp.inf)
        scnt[...] = jnp.zeros_like(scnt)

    deg = degp_ref[0] + degp_ref[1] + 1.0
    dinv = lax.rsqrt(deg)
    h2 = jnp.maximum(dinv * (aggp_ref[...] + g2_ref[...]) + b2_ref[...], 0.0)

    bt = batch_ref[...]
    oh = (bt == lax.broadcasted_iota(jnp.int32, (1, _B), 1)).astype(jnp.float32)
    dn = (((0,), (0,)), ((), ()))
    ssum[...] += lax.dot_general(oh, h2, dn, preferred_element_type=jnp.float32)
    scnt[...] += lax.dot_general(oh, jnp.ones((_RB, 1), jnp.float32), dn,
                                 preferred_element_type=jnp.float32)

    bmin = jnp.min(bt)
    bmax = jnp.max(bt)
    for b in range(_B):
        @pl.when((bmin <= b) & (b <= bmax))
        def _seg(b=b):
            m = bt == b
            contrib = jnp.max(jnp.where(m, h2, -jnp.inf), axis=0, keepdims=True)
            smax[b:b + 1, :] = jnp.maximum(smax[b:b + 1, :], contrib)

    @pl.when(i == pl.num_programs(0) - 1)
    def _fin():
        gmp = smax[...]
        gmp = jnp.where(gmp == -jnp.inf, 0.0, gmp)
        gap = ssum[...] / jnp.maximum(scnt[...], 1.0)
        z = (jnp.dot(gmp, m1a_ref[...], preferred_element_type=jnp.float32)
             + jnp.dot(gap, m1b_ref[...], preferred_element_type=jnp.float32)
             + jnp.dot(rho_ref[...], m1c_ref[...], preferred_element_type=jnp.float32)
             + mb1_ref[...])
        z = jnp.maximum(z, 0.0)
        z = jnp.maximum(jnp.dot(z, m2_ref[...], preferred_element_type=jnp.float32)
                        + mb2_ref[...], 0.0)
        out_ref[...] = (jnp.dot(z, m3_ref[...], preferred_element_type=jnp.float32)
                        + mb3_ref[...])


_tc3 = pl.pallas_call(
    _tc3_body,
    grid=(_NBLK,),
    in_specs=[
        pl.BlockSpec((_RB, _H), lambda i: (i, 0)),
        pl.BlockSpec((_RB, _H), lambda i: (i, 0)),
        pl.BlockSpec((_NC, _RB, 1), lambda i: (0, i, 0)),
        pl.BlockSpec((_RB, 1), lambda i: (i, 0)),
        pl.BlockSpec((1, _H), lambda i: (0, 0)),
        pl.BlockSpec((_B, 1), lambda i: (0, 0)),
        pl.BlockSpec((_H, _H), lambda i: (0, 0)),
        pl.BlockSpec((_H, _H), lambda i: (0, 0)),
        pl.BlockSpec((1, _H), lambda i: (0, 0)),
        pl.BlockSpec((1, _H), lambda i: (0, 0)),
        pl.BlockSpec((_H, _H), lambda i: (0, 0)),
        pl.BlockSpec((1, _H), lambda i: (0, 0)),
        pl.BlockSpec((_H, _OUT), lambda i: (0, 0)),
        pl.BlockSpec((1, _OUT), lambda i: (0, 0)),
    ],
    out_specs=pl.BlockSpec((_B, _OUT), lambda i: (0, 0)),
    out_shape=jax.ShapeDtypeStruct((_B, _OUT), jnp.float32),
    scratch_shapes=[
        pltpu.VMEM((_B, _H), jnp.float32),
        pltpu.VMEM((_B, _H), jnp.float32),
        pltpu.VMEM((_B, 1), jnp.float32),
    ],
)


def kernel(x, edge_index, edge_attr, batch, rho, W1, b1, W2, b2,
           M1, mb1, M2, mb2, M3, mb3):
    xp = jnp.pad(x, ((0, _NP - _N), (0, 0)))
    batch_p = jnp.pad(batch, (0, _NP - _N), constant_values=_B).reshape(_NP, 1)
    pad = _EPAD - _E
    apad = jnp.arange(pad, dtype=jnp.int32)
    src = jnp.concatenate([edge_index[0], apad % _N])
    dst = jnp.concatenate([edge_index[1], _N + apad % (_NP - _N)])
    ew = jnp.concatenate([edge_attr, jnp.zeros((pad,), jnp.float32)])
    srcr = src.reshape(_NS, _ET)
    dstr = dst.reshape(_NS, _KW, _CH)
    ewr = ew.reshape(_NS, _ET)
    zn = jnp.zeros((_NP,), jnp.float32)
    znd = jnp.zeros((_NP, _HF), jnp.float32)
    rho_c = rho.reshape(_B, 1)
    m1a = M1[:_H]
    m1b = M1[_H:2 * _H]
    m1c = M1[2 * _H:]

    degp = _deg_kernel(dstr, ewr, zn).reshape(_NC, _NP, 1)
    g1 = _tc1(xp, degp, W1)
    aggp1 = _agg_kernel(g1.reshape(_NC * _NP, _HF), srcr, dstr, ewr, znd)
    g2 = _tc2(aggp1, g1, degp, W2, b1.reshape(1, _H))
    aggp2 = _agg_kernel(g2.reshape(_NC * _NP, _HF), srcr, dstr, ewr, znd)
    return _tc3(aggp2, g2, degp, batch_p, b2.reshape(1, _H), rho_c,
                m1a, m1b, m1c, mb1.reshape(1, _H), M2, mb2.reshape(1, _H),
                M3, mb3.reshape(1, _OUT))
